# Initial kernel scaffold; baseline (speedup 1.0000x reference)
import jax
import jax.numpy as jnp
from jax.experimental import pallas as pl


def kernel(x, edge_index, edge_weight, batch, Wp, bp, Wc0, bc0, g0, be0, Wc1, bc1, g1, be1, Wc2, bc2, g2, be2, Wq, bq, Wo, bo):
    raise NotImplementedError("write your pallas kernel here")


# speedup vs baseline: 2.2045x; 2.2045x over previous
import functools

import jax
import jax.numpy as jnp
from jax import lax
from jax.experimental import pallas as pl
from jax.experimental.pallas import tpu as pltpu

N = 10000
E = 320000
G = 64
F0 = 128
F = 256
R = 1000
NBLK = N // R
EPS = 1e-5


def _blk_rows(i):
    return (i, 0)


def _fixed(i):
    return (0, 0)


def _pre_body(x_ref, wpt_ref, bp_ref, w0t_ref, deg_ref, hp_ref, dinv_ref):
    pre = jnp.maximum(
        jnp.dot(x_ref[...], wpt_ref[...], preferred_element_type=jnp.float32)
        + bp_ref[...],
        0.0,
    )
    dinv = lax.rsqrt(deg_ref[...])
    h = jnp.dot(pre, w0t_ref[...], preferred_element_type=jnp.float32)
    hp_ref[...] = dinv * h
    dinv_ref[...] = dinv


def _pre_call(x, wpt, bp, w0t, deg2d):
    return pl.pallas_call(
        _pre_body,
        grid=(NBLK,),
        in_specs=[
            pl.BlockSpec((R, F0), _blk_rows),
            pl.BlockSpec((F0, F), _fixed),
            pl.BlockSpec((1, F), _fixed),
            pl.BlockSpec((F, F), _fixed),
            pl.BlockSpec((R, 1), _blk_rows),
        ],
        out_specs=[
            pl.BlockSpec((R, F), _blk_rows),
            pl.BlockSpec((R, 1), _blk_rows),
        ],
        out_shape=[
            jax.ShapeDtypeStruct((N, F), jnp.float32),
            jax.ShapeDtypeStruct((N, 1), jnp.float32),
        ],
    )(x, wpt, bp, w0t, deg2d)


def _stats_body(s_ref, hp_ref, dinv_ref, u_ref, st_ref):
    i = pl.program_id(0)
    u = dinv_ref[...] * (s_ref[...] + 2.0 * hp_ref[...])
    u_ref[...] = u
    s1 = jnp.sum(u, axis=0, keepdims=True)
    s2 = jnp.sum(u * u, axis=0, keepdims=True)
    blk = jnp.concatenate([s1, s2], axis=0)

    @pl.when(i == 0)
    def _():
        st_ref[...] = blk

    @pl.when(i > 0)
    def _():
        st_ref[...] = st_ref[...] + blk


def _stats_call(s, hp, dinv2d):
    return pl.pallas_call(
        _stats_body,
        grid=(NBLK,),
        in_specs=[
            pl.BlockSpec((R, F), _blk_rows),
            pl.BlockSpec((R, F), _blk_rows),
            pl.BlockSpec((R, 1), _blk_rows),
        ],
        out_specs=[
            pl.BlockSpec((R, F), _blk_rows),
            pl.BlockSpec((2, F), _fixed),
        ],
        out_shape=[
            jax.ShapeDtypeStruct((N, F), jnp.float32),
            jax.ShapeDtypeStruct((2, F), jnp.float32),
        ],
    )(s, hp, dinv2d)


def _bn_mm_body(u_ref, st_ref, g_ref, be_ref, wt_ref, dinv_ref, hp_ref):
    m = st_ref[0:1, :] * (1.0 / N)
    var = st_ref[1:2, :] * (1.0 / N) - m * m
    scale = g_ref[...] * lax.rsqrt(var + EPS)
    z = jnp.maximum((u_ref[...] - m) * scale + be_ref[...], 0.0)
    h = jnp.dot(z, wt_ref[...], preferred_element_type=jnp.float32)
    hp_ref[...] = dinv_ref[...] * h


def _bn_mm_call(u, st, g, be, wt, dinv2d):
    return pl.pallas_call(
        _bn_mm_body,
        grid=(NBLK,),
        in_specs=[
            pl.BlockSpec((R, F), _blk_rows),
            pl.BlockSpec((2, F), _fixed),
            pl.BlockSpec((1, F), _fixed),
            pl.BlockSpec((1, F), _fixed),
            pl.BlockSpec((F, F), _fixed),
            pl.BlockSpec((R, 1), _blk_rows),
        ],
        out_specs=pl.BlockSpec((R, F), _blk_rows),
        out_shape=jax.ShapeDtypeStruct((N, F), jnp.float32),
    )(u, st, g, be, wt, dinv2d)


def _bn_body(u_ref, st_ref, g_ref, be_ref, z_ref):
    m = st_ref[0:1, :] * (1.0 / N)
    var = st_ref[1:2, :] * (1.0 / N) - m * m
    scale = g_ref[...] * lax.rsqrt(var + EPS)
    z_ref[...] = jnp.maximum((u_ref[...] - m) * scale + be_ref[...], 0.0)


def _bn_call(u, st, g, be):
    return pl.pallas_call(
        _bn_body,
        grid=(NBLK,),
        in_specs=[
            pl.BlockSpec((R, F), _blk_rows),
            pl.BlockSpec((2, F), _fixed),
            pl.BlockSpec((1, F), _fixed),
            pl.BlockSpec((1, F), _fixed),
        ],
        out_specs=pl.BlockSpec((R, F), _blk_rows),
        out_shape=jax.ShapeDtypeStruct((N, F), jnp.float32),
    )(u, st, g, be)


def _pool_body(z_ref, b_ref, sums_ref, cnt_ref):
    i = pl.program_id(0)
    gids = lax.broadcasted_iota(jnp.int32, (1, G), 1)
    onehot = (b_ref[...] == gids).astype(jnp.float32)
    psum = jnp.dot(onehot.T, z_ref[...], preferred_element_type=jnp.float32)
    pcnt = jnp.sum(onehot, axis=0, keepdims=True)

    @pl.when(i == 0)
    def _():
        sums_ref[...] = psum
        cnt_ref[...] = pcnt

    @pl.when(i > 0)
    def _():
        sums_ref[...] = sums_ref[...] + psum
        cnt_ref[...] = cnt_ref[...] + pcnt


def _pool_call(z, batch2d):
    return pl.pallas_call(
        _pool_body,
        grid=(NBLK,),
        in_specs=[
            pl.BlockSpec((R, F), _blk_rows),
            pl.BlockSpec((R, 1), _blk_rows),
        ],
        out_specs=[
            pl.BlockSpec((G, F), _fixed),
            pl.BlockSpec((1, G), _fixed),
        ],
        out_shape=[
            jax.ShapeDtypeStruct((G, F), jnp.float32),
            jax.ShapeDtypeStruct((1, G), jnp.float32),
        ],
    )(z, batch2d)


def _head_body(sums_ref, cnt_ref, wqt_ref, bq_ref, wot_ref, bo_ref, out_ref):
    cnt = jnp.maximum(cnt_ref[...], 1.0)
    pooled = sums_ref[...] / cnt.T
    r = jnp.maximum(
        jnp.dot(pooled, wqt_ref[...], preferred_element_type=jnp.float32)
        + bq_ref[...],
        0.0,
    )
    out_ref[...] = (
        jnp.dot(r, wot_ref[...], preferred_element_type=jnp.float32) + bo_ref[...]
    )


def _head_call(sums, cnt, wqt, bq, wot, bo):
    return pl.pallas_call(
        _head_body,
        out_shape=jax.ShapeDtypeStruct((G, 1), jnp.float32),
    )(sums, cnt, wqt, bq, wot, bo)


def _deg_jax(dst, ew):
    return jnp.zeros((N,), jnp.float32).at[dst].add(ew) + 2.0


def _scatter_jax(hp, src, dst, ew):
    return jnp.zeros((N, F), jnp.float32).at[dst].add(hp[src] * ew[:, None])


def kernel(x, edge_index, edge_weight, batch, Wp, bp, Wc0, bc0, g0, be0,
           Wc1, bc1, g1, be1, Wc2, bc2, g2, be2, Wq, bq, Wo, bo):
    src, dst = edge_index[0], edge_index[1]
    deg = _deg_jax(dst, edge_weight)

    hp = _pre_call(x, Wp.T, bp[None, :], Wc0.T, deg[:, None])
    hp, dinv2d = hp[0], hp[1]

    for Wnext, g, be in ((Wc1, g0, be0), (Wc2, g1, be1)):
        s = _scatter_jax(hp, src, dst, edge_weight)
        u, st = _stats_call(s, hp, dinv2d)
        hp = _bn_mm_call(u, st, g[None, :], be[None, :], Wnext.T, dinv2d)

    s = _scatter_jax(hp, src, dst, edge_weight)
    u, st = _stats_call(s, hp, dinv2d)
    z = _bn_call(u, st, g2[None, :], be2[None, :])

    sums, cnt = _pool_call(z, batch[:, None])
    return _head_call(sums, cnt, Wq.T, bq[None, :], Wo.T, bo[None, :])


# speedup vs baseline: 3.8324x; 1.7385x over previous
import functools

import jax
import jax.numpy as jnp
from jax import lax
from jax.experimental import pallas as pl
from jax.experimental.pallas import tpu as pltpu
from jax.experimental.pallas import tpu_sc as plsc

N = 10000
E = 320000
G = 64
F0 = 128
F = 256
R = 1000
NBLK = N // R
EPS = 1e-5

import dataclasses


def _sc_params():
    cp = pltpu.CompilerParams()
    if "needs_layout_passes" in pltpu.CompilerParams.__dataclass_fields__:
        cp = dataclasses.replace(cp, needs_layout_passes=False)
    return cp


NC = 2
NS = 16
L = 16
NW = NC * NS
TROWS = 320
NPAD = NW * TROWS
AROWS = TROWS + 8
GARB = TROWS + 2
C = 128
K = E // C
NB = 48


def _sc_mesh():
    return plsc.VectorSubcoreMesh(core_axis_name="c", subcore_axis_name="s")


def _deg_sc(dsts, ews, bnd, zeros):
    @functools.partial(
        pl.kernel,
        out_type=jax.ShapeDtypeStruct((NPAD,), jnp.float32),
        mesh=_sc_mesh(),
        compiler_params=_sc_params(),
        scratch_types=[
            pltpu.VMEM((AROWS,), jnp.float32),
            pltpu.VMEM((NB,), jnp.int32),
            pltpu.VMEM((C,), jnp.int32),
            pltpu.VMEM((C,), jnp.float32),
            pltpu.VMEM((C,), jnp.int32),
        ],
    )
    def k(dst_hbm, ew_hbm, bnd_hbm, z_hbm, deg_hbm, acc, bvec, dstv, ewv, ldst):
        w = lax.axis_index("c") * NS + lax.axis_index("s")
        base = w * TROWS
        lanes = lax.iota(jnp.int32, L)
        m0 = lanes == 0
        pltpu.sync_copy(z_hbm.at[pl.ds(0, AROWS)], acc)
        pltpu.sync_copy(bnd_hbm, bvec)
        lo = jnp.max(plsc.load_gather(bvec, [jnp.full((L,), w, jnp.int32)]))
        hi = jnp.max(plsc.load_gather(bvec, [jnp.full((L,), w + 1, jnp.int32)]))

        @pl.loop(lo >> 7, (hi + C - 1) >> 7)
        def _(kk):
            off = kk * C
            pltpu.sync_copy(dst_hbm.at[pl.ds(off, C)], dstv)
            pltpu.sync_copy(ew_hbm.at[pl.ds(off, C)], ewv)
            for i in range(C // L):
                dvec = dstv[pl.ds(i * L, L)]
                inr = (dvec >= base) & (dvec < base + TROWS)
                ldst.at[pl.ds(i * L, L)][...] = jnp.where(inr, dvec - base, GARB)

            @pl.loop(0, C)
            def _(r):
                rr = jnp.full((L,), r, jnp.int32)
                lrow = plsc.load_gather(ldst, [rr])
                wv = plsc.load_gather(ewv, [rr])
                plsc.addupdate_scatter(acc, [lrow], wv, mask=m0)

        pltpu.sync_copy(acc.at[pl.ds(0, TROWS)], deg_hbm.at[pl.ds(base, TROWS)])

    return k(dsts, ews, bnd, zeros)


def _scatter_sc(hp, srcs, dsts, ews, bnd, zeros):
    @functools.partial(
        pl.kernel,
        out_type=jax.ShapeDtypeStruct((NPAD * F,), jnp.float32),
        mesh=_sc_mesh(),
        compiler_params=_sc_params(),
        scratch_types=[
            pltpu.VMEM((AROWS * F,), jnp.float32),
            pltpu.VMEM((NB,), jnp.int32),
            pltpu.VMEM((C,), jnp.int32),
            pltpu.VMEM((C,), jnp.int32),
            pltpu.VMEM((C,), jnp.float32),
            pltpu.VMEM((C,), jnp.int32),
            pltpu.VMEM((C, F), jnp.float32),
            pltpu.SemaphoreType.DMA,
        ],
    )
    def k(hp_hbm, src_hbm, dst_hbm, ew_hbm, bnd_hbm, z_hbm, s_hbm, acc, bvec,
          sidx, dstv, ewv, ldst, rows, sem):
        w = lax.axis_index("c") * NS + lax.axis_index("s")
        base = w * TROWS
        lanes = lax.iota(jnp.int32, L)
        pltpu.sync_copy(z_hbm, acc)
        pltpu.sync_copy(bnd_hbm, bvec)
        lo = jnp.max(plsc.load_gather(bvec, [jnp.full((L,), w, jnp.int32)]))
        hi = jnp.max(plsc.load_gather(bvec, [jnp.full((L,), w + 1, jnp.int32)]))

        @pl.loop(lo >> 7, (hi + C - 1) >> 7)
        def _(kk):
            off = kk * C
            pltpu.sync_copy(src_hbm.at[pl.ds(off, C)], sidx)
            pltpu.sync_copy(dst_hbm.at[pl.ds(off, C)], dstv)
            pltpu.sync_copy(ew_hbm.at[pl.ds(off, C)], ewv)
            pltpu.async_copy(hp_hbm.at[sidx], rows, sem).wait()
            for i in range(C // L):
                dvec = dstv[pl.ds(i * L, L)]
                inr = (dvec >= base) & (dvec < base + TROWS)
                ldst.at[pl.ds(i * L, L)][...] = jnp.where(inr, dvec - base, GARB)

            @pl.loop(0, C)
            def _(r):
                rr = jnp.full((L,), r, jnp.int32)
                lrow = plsc.load_gather(ldst, [rr])
                wv = plsc.load_gather(ewv, [rr])
                rbase = lrow * F
                for f in range(F // L):
                    addr = rbase + (f * L) + lanes
                    val = rows.at[r, pl.ds(f * L, L)][...] * wv
                    plsc.addupdate_scatter(acc, [addr], val)

        pltpu.sync_copy(
            acc.at[pl.ds(0, TROWS * F)], s_hbm.at[pl.ds(base * F, TROWS * F)]
        )

    return k(hp, srcs, dsts, ews, bnd, zeros)


def _blk_rows(i):
    return (i, 0)


def _fixed(i):
    return (0, 0)


def _pre_body(x_ref, wpt_ref, bp_ref, w0t_ref, deg_ref, hp_ref, dinv_ref):
    pre = jnp.maximum(
        jnp.dot(x_ref[...], wpt_ref[...], preferred_element_type=jnp.float32)
        + bp_ref[...],
        0.0,
    )
    dinv = lax.rsqrt(deg_ref[...] + 2.0)
    h = jnp.dot(pre, w0t_ref[...], preferred_element_type=jnp.float32)
    hp_ref[...] = dinv * h
    dinv_ref[...] = dinv


def _pre_call(x, wpt, bp, w0t, deg2d):
    return pl.pallas_call(
        _pre_body,
        grid=(NBLK,),
        in_specs=[
            pl.BlockSpec((R, F0), _blk_rows),
            pl.BlockSpec((F0, F), _fixed),
            pl.BlockSpec((1, F), _fixed),
            pl.BlockSpec((F, F), _fixed),
            pl.BlockSpec((R, 1), _blk_rows),
        ],
        out_specs=[
            pl.BlockSpec((R, F), _blk_rows),
            pl.BlockSpec((R, 1), _blk_rows),
        ],
        out_shape=[
            jax.ShapeDtypeStruct((N, F), jnp.float32),
            jax.ShapeDtypeStruct((N, 1), jnp.float32),
        ],
    )(x, wpt, bp, w0t, deg2d)


def _stats_body(s_ref, hp_ref, dinv_ref, u_ref, st_ref):
    i = pl.program_id(0)
    u = dinv_ref[...] * (s_ref[...] + 2.0 * hp_ref[...])
    u_ref[...] = u
    s1 = jnp.sum(u, axis=0, keepdims=True)
    s2 = jnp.sum(u * u, axis=0, keepdims=True)
    blk = jnp.concatenate([s1, s2], axis=0)

    @pl.when(i == 0)
    def _():
        st_ref[...] = blk

    @pl.when(i > 0)
    def _():
        st_ref[...] = st_ref[...] + blk


def _stats_call(s, hp, dinv2d):
    return pl.pallas_call(
        _stats_body,
        grid=(NBLK,),
        in_specs=[
            pl.BlockSpec((R, F), _blk_rows),
            pl.BlockSpec((R, F), _blk_rows),
            pl.BlockSpec((R, 1), _blk_rows),
        ],
        out_specs=[
            pl.BlockSpec((R, F), _blk_rows),
            pl.BlockSpec((2, F), _fixed),
        ],
        out_shape=[
            jax.ShapeDtypeStruct((N, F), jnp.float32),
            jax.ShapeDtypeStruct((2, F), jnp.float32),
        ],
    )(s, hp, dinv2d)


def _bn_mm_body(u_ref, st_ref, g_ref, be_ref, wt_ref, dinv_ref, hp_ref):
    m = st_ref[0:1, :] * (1.0 / N)
    var = st_ref[1:2, :] * (1.0 / N) - m * m
    scale = g_ref[...] * lax.rsqrt(var + EPS)
    z = jnp.maximum((u_ref[...] - m) * scale + be_ref[...], 0.0)
    h = jnp.dot(z, wt_ref[...], preferred_element_type=jnp.float32)
    hp_ref[...] = dinv_ref[...] * h


def _bn_mm_call(u, st, g, be, wt, dinv2d):
    return pl.pallas_call(
        _bn_mm_body,
        grid=(NBLK,),
        in_specs=[
            pl.BlockSpec((R, F), _blk_rows),
            pl.BlockSpec((2, F), _fixed),
            pl.BlockSpec((1, F), _fixed),
            pl.BlockSpec((1, F), _fixed),
            pl.BlockSpec((F, F), _fixed),
            pl.BlockSpec((R, 1), _blk_rows),
        ],
        out_specs=pl.BlockSpec((R, F), _blk_rows),
        out_shape=jax.ShapeDtypeStruct((N, F), jnp.float32),
    )(u, st, g, be, wt, dinv2d)


def _bn_body(u_ref, st_ref, g_ref, be_ref, z_ref):
    m = st_ref[0:1, :] * (1.0 / N)
    var = st_ref[1:2, :] * (1.0 / N) - m * m
    scale = g_ref[...] * lax.rsqrt(var + EPS)
    z_ref[...] = jnp.maximum((u_ref[...] - m) * scale + be_ref[...], 0.0)


def _bn_call(u, st, g, be):
    return pl.pallas_call(
        _bn_body,
        grid=(NBLK,),
        in_specs=[
            pl.BlockSpec((R, F), _blk_rows),
            pl.BlockSpec((2, F), _fixed),
            pl.BlockSpec((1, F), _fixed),
            pl.BlockSpec((1, F), _fixed),
        ],
        out_specs=pl.BlockSpec((R, F), _blk_rows),
        out_shape=jax.ShapeDtypeStruct((N, F), jnp.float32),
    )(u, st, g, be)


def _pool_body(z_ref, b_ref, sums_ref, cnt_ref):
    i = pl.program_id(0)
    gids = lax.broadcasted_iota(jnp.int32, (1, G), 1)
    onehot = (b_ref[...] == gids).astype(jnp.float32)
    psum = jnp.dot(onehot.T, z_ref[...], preferred_element_type=jnp.float32)
    pcnt = jnp.sum(onehot, axis=0, keepdims=True)

    @pl.when(i == 0)
    def _():
        sums_ref[...] = psum
        cnt_ref[...] = pcnt

    @pl.when(i > 0)
    def _():
        sums_ref[...] = sums_ref[...] + psum
        cnt_ref[...] = cnt_ref[...] + pcnt


def _pool_call(z, batch2d):
    return pl.pallas_call(
        _pool_body,
        grid=(NBLK,),
        in_specs=[
            pl.BlockSpec((R, F), _blk_rows),
            pl.BlockSpec((R, 1), _blk_rows),
        ],
        out_specs=[
            pl.BlockSpec((G, F), _fixed),
            pl.BlockSpec((1, G), _fixed),
        ],
        out_shape=[
            jax.ShapeDtypeStruct((G, F), jnp.float32),
            jax.ShapeDtypeStruct((1, G), jnp.float32),
        ],
    )(z, batch2d)


def _head_body(sums_ref, cnt_ref, wqt_ref, bq_ref, wot_ref, bo_ref, out_ref):
    cnt = jnp.maximum(cnt_ref[...], 1.0)
    pooled = sums_ref[...] / cnt.T
    r = jnp.maximum(
        jnp.dot(pooled, wqt_ref[...], preferred_element_type=jnp.float32)
        + bq_ref[...],
        0.0,
    )
    out_ref[...] = (
        jnp.dot(r, wot_ref[...], preferred_element_type=jnp.float32) + bo_ref[...]
    )


def _head_call(sums, cnt, wqt, bq, wot, bo):
    return pl.pallas_call(
        _head_body,
        out_shape=jax.ShapeDtypeStruct((G, 1), jnp.float32),
    )(sums, cnt, wqt, bq, wot, bo)


def kernel(x, edge_index, edge_weight, batch, Wp, bp, Wc0, bc0, g0, be0,
           Wc1, bc1, g1, be1, Wc2, bc2, g2, be2, Wq, bq, Wo, bo):
    src, dst = edge_index[0], edge_index[1]
    perm = jnp.argsort(dst)
    srcs, dsts, ews = src[perm], dst[perm], edge_weight[perm]
    bnd = jnp.searchsorted(
        dsts, jnp.arange(0, NB * TROWS, TROWS, dtype=jnp.int32)
    ).astype(jnp.int32)
    zeros = jnp.zeros((AROWS * F,), jnp.float32)

    deg = _deg_sc(dsts, ews, bnd, zeros)[:N]

    hp, dinv2d = _pre_call(x, Wp.T, bp[None, :], Wc0.T, deg[:, None])

    for Wnext, g, be in ((Wc1, g0, be0), (Wc2, g1, be1)):
        sflat = _scatter_sc(hp, srcs, dsts, ews, bnd, zeros)
        u, st = _stats_call(sflat.reshape(NPAD, F)[:N], hp, dinv2d)
        hp = _bn_mm_call(u, st, g[None, :], be[None, :], Wnext.T, dinv2d)

    sflat = _scatter_sc(hp, srcs, dsts, ews, bnd, zeros)
    u, st = _stats_call(sflat.reshape(NPAD, F)[:N], hp, dinv2d)
    z = _bn_call(u, st, g2[None, :], be2[None, :])

    sums, cnt = _pool_call(z, batch[:, None])
    return _head_call(sums, cnt, Wq.T, bq[None, :], Wo.T, bo[None, :])
